# baseline (speedup 1.0000x reference)
import functools

import jax
import jax.numpy as jnp
import numpy as np
from jax import lax
from jax.experimental import pallas as pl
from jax.experimental.pallas import tpu as pltpu
from jax.experimental.pallas import tpu_sc as plsc

N, E, DIN, D, CIN, B, G, H = 10000, 320000, 128, 256, 512, 1024, 256, 4

NW = 32
C1 = 80
EC1 = E // NW
NCH1 = EC1 // C1
C2 = 32
EC2 = E // 16
NCH2 = EC2 // C2
W2 = 144
NSTR = N // 16
ZR = 125

_mesh = plsc.VectorSubcoreMesh(core_axis_name="c", subcore_axis_name="s")


def _iota16():
    return lax.iota(jnp.int32, 16)


def _splat(vec, i):
    return vec.at[jnp.full((16,), i, jnp.int32)].get(mode="promise_in_bounds")


def _lanesum(a):
    iota = _iota16()
    for sh in (8, 4, 2, 1):
        a = a + a.at[iota ^ sh].get(mode="promise_in_bounds")
    return a


def _lanemax(a):
    iota = _iota16()
    for sh in (8, 4, 2, 1):
        a = jnp.maximum(a, a.at[iota ^ sh].get(mode="promise_in_bounds"))
    return a


def _sc_logits_body(q_hbm, k_hbm, src2_hbm, dst2_hbm, log_hbm, max_hbm,
                    didx_all, sidx_all, qbuf, kbuf, lbuf, mbuf,
                    gq0, gq1, gk0, gk1, st0, st1):
    c = lax.axis_index("c")
    s = lax.axis_index("s")
    wid = s * 2 + c
    rbase = wid * NCH1
    gq = (gq0, gq1)
    gk = (gk0, gk1)
    st = (st0, st1)
    iota = _iota16()

    pltpu.sync_copy(dst2_hbm.at[pl.ds(rbase, NCH1)], didx_all)
    pltpu.sync_copy(src2_hbm.at[pl.ds(rbase, NCH1)], sidx_all)

    def gstart(ci, b):
        pltpu.async_copy(q_hbm.at[didx_all.at[ci]], qbuf.at[b], gq[b])
        pltpu.async_copy(k_hbm.at[sidx_all.at[ci]], kbuf.at[b], gk[b])

    def gwait(ci, b):
        pltpu.make_async_copy(q_hbm.at[didx_all.at[ci]], qbuf.at[b], gq[b]).wait()
        pltpu.make_async_copy(k_hbm.at[sidx_all.at[ci]], kbuf.at[b], gk[b]).wait()

    def process(ci, b, mrun):
        eb4 = (rbase + ci) * (C1 * 4)
        gwait(ci, b)

        @pl.when(ci >= 2)
        def _():
            pltpu.make_async_copy(
                lbuf.at[b], log_hbm.at[pl.ds(eb4, C1 * 4)], st[b]).wait()

        def quad_body(jq, mrun):
            lvec = jnp.zeros((16,), jnp.float32)
            for r in range(4):
                e = jq * 4 + r
                for h in range(H):
                    a = (qbuf[b, e, pl.ds(h * 64, 16)]
                         * kbuf[b, e, pl.ds(h * 64, 16)])
                    for j in range(1, 4):
                        o = h * 64 + j * 16
                        a = a + (qbuf[b, e, pl.ds(o, 16)]
                                 * kbuf[b, e, pl.ds(o, 16)])
                    sv = _lanesum(a) * 0.125
                    lvec = jnp.where(iota == 4 * r + h, sv, lvec)
            lbuf[b, pl.ds(jq * 16, 16)] = lvec
            return jnp.maximum(mrun, lvec)

        mrun = lax.fori_loop(0, C1 // 4, quad_body, mrun, unroll=10)
        pltpu.async_copy(lbuf.at[b], log_hbm.at[pl.ds(eb4, C1 * 4)], st[b])

        @pl.when(ci + 2 < NCH1)
        def _():
            gstart(ci + 2, b)

        return mrun

    gstart(0, 0)
    gstart(1, 1)

    def pair_body(t, mrun):
        mrun = process(2 * t, 0, mrun)
        mrun = process(2 * t + 1, 1, mrun)
        return mrun

    mrun = lax.fori_loop(0, NCH1 // 2, pair_body,
                         jnp.full((16,), -3.0e38, jnp.float32))
    if NCH1 % 2:
        mrun = process(NCH1 - 1, 0, mrun)
    pltpu.make_async_copy(lbuf.at[0], log_hbm.at[pl.ds(0, C1 * 4)], st[0]).wait()
    pltpu.make_async_copy(lbuf.at[1], log_hbm.at[pl.ds(0, C1 * 4)], st[1]).wait()
    mbuf[...] = mrun
    pltpu.sync_copy(mbuf, max_hbm.at[wid])


_sc_logits = functools.partial(
    pl.kernel,
    out_type=(jax.ShapeDtypeStruct((E * 4,), jnp.float32),
              jax.ShapeDtypeStruct((NW, 16), jnp.float32)),
    mesh=_mesh,
    compiler_params=pltpu.CompilerParams(use_tc_tiling_on_sc=False),
    scratch_types=[
        pltpu.VMEM((NCH1, C1), jnp.int32),
        pltpu.VMEM((NCH1, C1), jnp.int32),
        pltpu.VMEM((2, C1, D), jnp.float32),
        pltpu.VMEM((2, C1, D), jnp.float32),
        pltpu.VMEM((2, C1 * 4), jnp.float32),
        pltpu.VMEM((16,), jnp.float32),
        pltpu.SemaphoreType.DMA,
        pltpu.SemaphoreType.DMA,
        pltpu.SemaphoreType.DMA,
        pltpu.SemaphoreType.DMA,
        pltpu.SemaphoreType.DMA,
        pltpu.SemaphoreType.DMA,
    ],
)(_sc_logits_body)


def _sc_accum_body(ex_hbm, src2_hbm, dst2_hbm, vcat_hbm,
                   cvals_hbm, out_hbm,
                   acc, didx, sidx, didxs, vbufg, vbuf, exbuf, zbuf, cbuf,
                   ix0, ix1, gv0, gv1, ge0, ge1, sc0, sc1):
    c = lax.axis_index("c")
    s = lax.axis_index("s")
    ix = (ix0, ix1)
    gv = (gv0, gv1)
    ge = (ge0, ge1)
    scs = (sc0, sc1)
    pltpu.sync_copy(cvals_hbm.at[c], cbuf)
    c2v = cbuf[...]
    rowoff = c2v * (N // 2)
    iota = _iota16()
    rbase = s * NCH2

    zv = jnp.zeros((16,), jnp.float32)

    def zrow(r, _):
        for j in range(W2 // 16):
            zbuf[r, pl.ds(j * 16, 16)] = zv
        return 0

    lax.fori_loop(0, ZR, zrow, 0)
    for t in range(NSTR // ZR):
        pltpu.sync_copy(zbuf, acc.at[pl.ds(s * NSTR + t * ZR, ZR)])
    plsc.subcore_barrier()

    def idxstart(ci, b):
        pltpu.async_copy(dst2_hbm.at[rbase + ci], didx.at[b], ix[b])
        pltpu.async_copy(src2_hbm.at[rbase + ci], sidx.at[b], ix[b])

    def idxwait(ci, b):
        pltpu.make_async_copy(dst2_hbm.at[rbase + ci], didx.at[b], ix[b]).wait()
        pltpu.make_async_copy(src2_hbm.at[rbase + ci], sidx.at[b], ix[b]).wait()

    def gstart(ci, b):
        for j in range(C2 // 16):
            sidx[b, pl.ds(j * 16, 16)] = sidx[b, pl.ds(j * 16, 16)] + rowoff
        pltpu.async_copy(vcat_hbm.at[sidx.at[b]], vbufg.at[b], gv[b])
        pltpu.async_copy(
            ex_hbm.at[pl.ds((rbase + ci) * (C2 * 4), C2 * 4)],
            exbuf.at[b], ge[b])

    def process(ci, b):
        b2 = 1 - b

        @pl.when(ci + 1 < NCH2)
        def _():
            idxwait(ci + 1, b2)
            gstart(ci + 1, b2)

        pltpu.make_async_copy(
            vcat_hbm.at[sidx.at[b]], vbufg.at[b], gv[b]).wait()
        pltpu.make_async_copy(
            ex_hbm.at[pl.ds((rbase + ci) * (C2 * 4), C2 * 4)],
            exbuf.at[b], ge[b]).wait()

        @pl.when(ci >= 2)
        def _():
            pltpu.make_async_copy(
                vbuf.at[b], acc.at[didxs.at[b]], scs[b]).wait()

        for j in range(C2 // 16):
            didxs[b, pl.ds(j * 16, 16)] = didx[b, pl.ds(j * 16, 16)]

        @pl.when(ci + 2 < NCH2)
        def _():
            idxstart(ci + 2, b)

        def quad_body(jq, _):
            qv = exbuf[b, pl.ds(jq * 16, 16)]
            for r in range(4):
                e = jq * 4 + r
                i0 = jnp.full((16,), 4 * r, jnp.int32) + c2v
                m0 = qv.at[i0].get(mode="promise_in_bounds")
                m1 = qv.at[i0 + 1].get(mode="promise_in_bounds")
                for jv in range(8):
                    g = vbufg[b, e, pl.ds(jv * 16, 16)]
                    vbuf[b, e, pl.ds(jv * 16, 16)] = g * (m0 if jv < 4 else m1)
                tidx = jnp.where(iota < 4, iota + 4 * r, 0)
                tail = jnp.where(
                    iota < 4,
                    qv.at[tidx].get(mode="promise_in_bounds"),
                    0.0)
                vbuf[b, e, pl.ds(128, 16)] = tail
            return 0

        lax.fori_loop(0, C2 // 4, quad_body, 0, unroll=8)
        pltpu.async_copy(vbuf.at[b], acc.at[didxs.at[b]], scs[b], add=True)

    idxstart(0, 0)
    idxstart(1, 1)
    idxwait(0, 0)
    gstart(0, 0)

    def pair_body(t, _):
        process(2 * t, 0)
        process(2 * t + 1, 1)
        return 0

    lax.fori_loop(0, NCH2 // 2, pair_body, 0)
    if NCH2 % 2:
        process(NCH2 - 1, 0)
    pltpu.make_async_copy(vbuf.at[0], acc.at[didxs.at[0]], scs[0]).wait()
    pltpu.make_async_copy(vbuf.at[1], acc.at[didxs.at[1]], scs[1]).wait()
    plsc.subcore_barrier()

    for t in range(NSTR // ZR):
        start = s * NSTR + t * ZR
        pltpu.sync_copy(acc.at[pl.ds(start, ZR)],
                        out_hbm.at[pl.ds(c * N + start, ZR)])


_sc_accum = functools.partial(
    pl.kernel,
    out_type=jax.ShapeDtypeStruct((2 * N, W2), jnp.float32),
    mesh=_mesh,
    compiler_params=pltpu.CompilerParams(use_tc_tiling_on_sc=False),
    scratch_types=[
        pltpu.VMEM_SHARED((N, W2), jnp.float32),
        pltpu.VMEM((2, C2), jnp.int32),
        pltpu.VMEM((2, C2), jnp.int32),
        pltpu.VMEM((2, C2), jnp.int32),
        pltpu.VMEM((2, C2, 128), jnp.float32),
        pltpu.VMEM((2, C2, W2), jnp.float32),
        pltpu.VMEM((2, C2 * 4), jnp.float32),
        pltpu.VMEM((ZR, W2), jnp.float32),
        pltpu.VMEM((16,), jnp.int32),
        pltpu.SemaphoreType.DMA,
        pltpu.SemaphoreType.DMA,
        pltpu.SemaphoreType.DMA,
        pltpu.SemaphoreType.DMA,
        pltpu.SemaphoreType.DMA,
        pltpu.SemaphoreType.DMA,
        pltpu.SemaphoreType.DMA,
        pltpu.SemaphoreType.DMA,
    ],
)(_sc_accum_body)


def _exp_body(l_ref, m_ref, ex_ref):
    M = jnp.max(m_ref[:])
    ex_ref[:] = jnp.exp(l_ref[:] - M)


def _exp_tc(logits, tmax):
    ex = pl.pallas_call(
        _exp_body,
        out_shape=jax.ShapeDtypeStruct((E * 4 // 512, 512), jnp.float32),
    )(logits.reshape(E * 4 // 512, 512), tmax)
    return ex.reshape(E * 4)


_QB = 2000


def _qkv_body(x_ref, wq_ref, bq_ref, wk_ref, bk_ref, wv_ref, bv_ref,
              q_ref, k_ref, vcat_ref):
    x = x_ref[:]
    q_ref[:] = jnp.dot(x, wq_ref[:], preferred_element_type=jnp.float32) + bq_ref[:]
    k_ref[:] = jnp.dot(x, wk_ref[:], preferred_element_type=jnp.float32) + bk_ref[:]
    v = jnp.dot(x, wv_ref[:], preferred_element_type=jnp.float32) + bv_ref[:]
    vcat_ref[0] = v[:, :128]
    vcat_ref[1] = v[:, 128:]


def _qkv(x, Wq, bq, Wk, bk, Wv, bv):
    fi = x.shape[1]
    q, k, v2 = pl.pallas_call(
        _qkv_body,
        grid=(N // _QB,),
        in_specs=[
            pl.BlockSpec((_QB, fi), lambda i: (i, 0)),
            pl.BlockSpec((fi, D), lambda i: (0, 0)),
            pl.BlockSpec((1, D), lambda i: (0, 0)),
            pl.BlockSpec((fi, D), lambda i: (0, 0)),
            pl.BlockSpec((1, D), lambda i: (0, 0)),
            pl.BlockSpec((fi, D), lambda i: (0, 0)),
            pl.BlockSpec((1, D), lambda i: (0, 0)),
        ],
        out_specs=(
            pl.BlockSpec((_QB, D), lambda i: (i, 0)),
            pl.BlockSpec((_QB, D), lambda i: (i, 0)),
            pl.BlockSpec((2, _QB, 128), lambda i: (0, i, 0)),
        ),
        out_shape=(jax.ShapeDtypeStruct((N, D), jnp.float32),
                   jax.ShapeDtypeStruct((N, D), jnp.float32),
                   jax.ShapeDtypeStruct((2, N, 128), jnp.float32)),
    )(x, Wq, bq.reshape(1, D), Wk, bk.reshape(1, D), Wv, bv.reshape(1, D))
    return q, k, v2.reshape(2 * N, 128)


def _post_core(outc, g, beta):
    t = jnp.concatenate([outc[:N, :128], outc[N:, :128]], axis=1)
    den = outc[:N, 128:132]
    den_exp = jnp.concatenate(
        [jnp.broadcast_to(den[:, i:i + 1], (N, 64)) for i in range(4)], axis=1)
    t = jax.nn.relu(t / (den_exp + 1e-16))
    mu = jnp.mean(t, axis=0, keepdims=True)
    var = jnp.mean((t - mu) ** 2, axis=0, keepdims=True)
    return g * (t - mu) / jnp.sqrt(var + 1e-5) + beta


def _post_body(outc_ref, res_ref, g_ref, beta_ref, h_ref, *, use_res):
    h = _post_core(outc_ref[:], g_ref[:], beta_ref[:])
    if use_res:
        h = h + res_ref[:]
    h_ref[:] = h


def _post(outc, res, g, beta, use_res):
    return pl.pallas_call(
        functools.partial(_post_body, use_res=use_res),
        out_shape=jax.ShapeDtypeStruct((N, D), jnp.float32),
    )(outc, res, g.reshape(1, D), beta.reshape(1, D))


def _pool_body(h_ref, ib_ref, pool_ref):
    ib = ib_ref[:]
    gids = lax.broadcasted_iota(jnp.int32, (G, N), 0)
    onehot = (gids == ib).astype(jnp.float32)
    cnt = jnp.sum(onehot, axis=1, keepdims=True)
    h = h_ref[:]
    h_hi = h.astype(jnp.bfloat16).astype(jnp.float32)
    h_lo = h - h_hi
    s = (jnp.dot(onehot, h_hi, preferred_element_type=jnp.float32)
         + jnp.dot(onehot, h_lo, preferred_element_type=jnp.float32))
    pool_ref[:] = s / jnp.maximum(cnt, 1.0)


def _pool(h, ibatch):
    return pl.pallas_call(
        _pool_body,
        out_shape=jax.ShapeDtypeStruct((G, D), jnp.float32),
    )(h, ibatch.reshape(1, N))


def _cline_body(x_ref, wc_ref, bc_ref, w1_ref, b1_ref, w2_ref, b2_ref, o_ref):
    c = jnp.tanh(jnp.dot(x_ref[:], wc_ref[:], preferred_element_type=jnp.float32) + bc_ref[:])
    c = c + jax.nn.relu(jnp.dot(c, w1_ref[:], preferred_element_type=jnp.float32) + b1_ref[:])
    c = c + jax.nn.relu(jnp.dot(c, w2_ref[:], preferred_element_type=jnp.float32) + b2_ref[:])
    o_ref[:] = c


def _cline_mlp(cline_x, Wc, bc, Wc1, bc1, Wc2, bc2):
    return pl.pallas_call(
        _cline_body,
        out_shape=jax.ShapeDtypeStruct((B, D), jnp.float32),
    )(cline_x, Wc, bc.reshape(1, D), Wc1, bc1.reshape(1, D), Wc2, bc2.reshape(1, D))


def kernel(drug_x, drug_adj, ibatch, cline_x, Wq0, Wk0, Wv0, bq0, bk0, bv0, g0, beta0, Wq1, Wk1, Wv1, bq1, bk1, bv1, g1, beta1, Wq2, Wk2, Wv2, bq2, bk2, bv2, g2, beta2, Wc, bc, Wc1, bc1, Wc2, bc2):
    src = drug_adj[0].reshape(E // C1, C1)
    dst = drug_adj[1].reshape(E // C1, C1)
    srcb = drug_adj[0].reshape(E // C2, C2)
    dstb = drug_adj[1].reshape(E // C2, C2)
    cvals = jnp.repeat(jnp.array([[0], [2]], jnp.int32), 16, axis=1)

    q, k, vcat = _qkv(drug_x, Wq0, bq0, Wk0, bk0, Wv0, bv0)
    logits, tmax = _sc_logits(q, k, src, dst)
    outc = _sc_accum(_exp_tc(logits, tmax), srcb, dstb, vcat, cvals)
    h1 = _post(outc, drug_x, g0, beta0, use_res=False)
    q, k, vcat = _qkv(h1, Wq1, bq1, Wk1, bk1, Wv1, bv1)

    logits, tmax = _sc_logits(q, k, src, dst)
    outc = _sc_accum(_exp_tc(logits, tmax), srcb, dstb, vcat, cvals)
    h2 = _post(outc, h1, g1, beta1, use_res=True)
    q, k, vcat = _qkv(h2, Wq2, bq2, Wk2, bk2, Wv2, bv2)

    logits, tmax = _sc_logits(q, k, src, dst)
    outc = _sc_accum(_exp_tc(logits, tmax), srcb, dstb, vcat, cvals)
    h3 = _post(outc, h2, g2, beta2, use_res=True)

    drug_pool = _pool(h3, ibatch)
    c = _cline_mlp(cline_x, Wc, bc, Wc1, bc1, Wc2, bc2)
    return (drug_pool, c)


# speedup vs baseline: 1.5713x; 1.5713x over previous
import functools

import jax
import jax.numpy as jnp
import numpy as np
from jax import lax
from jax.experimental import pallas as pl
from jax.experimental.pallas import tpu as pltpu
from jax.experimental.pallas import tpu_sc as plsc

N, E, DIN, D, CIN, B, G, H = 10000, 320000, 128, 256, 512, 1024, 256, 4

NW = 32
C1 = 80
EC1 = E // NW
NCH1 = EC1 // C1
C2 = 32
EC2 = E // 16
NCH2 = EC2 // C2
W2 = 144
NSTR = N // 16
ZR = 125

_mesh = plsc.VectorSubcoreMesh(core_axis_name="c", subcore_axis_name="s")


def _iota16():
    return lax.iota(jnp.int32, 16)


def _splat(vec, i):
    return vec.at[jnp.full((16,), i, jnp.int32)].get(mode="promise_in_bounds")


def _lanesum(a):
    iota = _iota16()
    for sh in (8, 4, 2, 1):
        a = a + a.at[iota ^ sh].get(mode="promise_in_bounds")
    return a


def _lanemax(a):
    iota = _iota16()
    for sh in (8, 4, 2, 1):
        a = jnp.maximum(a, a.at[iota ^ sh].get(mode="promise_in_bounds"))
    return a


def _sc_logits_body(q_hbm, k_hbm, src2_hbm, dst2_hbm, log_hbm, max_hbm,
                    didx_all, sidx_all, qbuf, kbuf, lbuf, mbuf,
                    gq0, gq1, gk0, gk1, st0, st1):
    c = lax.axis_index("c")
    s = lax.axis_index("s")
    wid = s * 2 + c
    rbase = wid * NCH1
    gq = (gq0, gq1)
    gk = (gk0, gk1)
    st = (st0, st1)
    iota = _iota16()

    pltpu.sync_copy(dst2_hbm.at[pl.ds(rbase, NCH1)], didx_all)
    pltpu.sync_copy(src2_hbm.at[pl.ds(rbase, NCH1)], sidx_all)

    def gstart(ci, b):
        pltpu.async_copy(q_hbm.at[didx_all.at[ci]], qbuf.at[b], gq[b])
        pltpu.async_copy(k_hbm.at[sidx_all.at[ci]], kbuf.at[b], gk[b])

    def gwait(ci, b):
        pltpu.make_async_copy(q_hbm.at[didx_all.at[ci]], qbuf.at[b], gq[b]).wait()
        pltpu.make_async_copy(k_hbm.at[sidx_all.at[ci]], kbuf.at[b], gk[b]).wait()

    def process(ci, b, mrun):
        eb4 = (rbase + ci) * (C1 * 4)
        gwait(ci, b)

        @pl.when(ci >= 2)
        def _():
            pltpu.make_async_copy(
                lbuf.at[b], log_hbm.at[pl.ds(eb4, C1 * 4)], st[b]).wait()

        def quad_body(jq, mrun):
            lvec = jnp.zeros((16,), jnp.float32)
            for r in range(4):
                e = jq * 4 + r
                for h in range(H):
                    a = (qbuf[b, e, pl.ds(h * 64, 16)]
                         * kbuf[b, e, pl.ds(h * 64, 16)])
                    for j in range(1, 4):
                        o = h * 64 + j * 16
                        a = a + (qbuf[b, e, pl.ds(o, 16)]
                                 * kbuf[b, e, pl.ds(o, 16)])
                    sv = _lanesum(a) * 0.125
                    lvec = jnp.where(iota == 4 * r + h, sv, lvec)
            lbuf[b, pl.ds(jq * 16, 16)] = lvec
            return jnp.maximum(mrun, lvec)

        mrun = lax.fori_loop(0, C1 // 4, quad_body, mrun, unroll=4)
        pltpu.async_copy(lbuf.at[b], log_hbm.at[pl.ds(eb4, C1 * 4)], st[b])

        @pl.when(ci + 2 < NCH1)
        def _():
            gstart(ci + 2, b)

        return mrun

    gstart(0, 0)
    gstart(1, 1)

    def pair_body(t, mrun):
        mrun = process(2 * t, 0, mrun)
        mrun = process(2 * t + 1, 1, mrun)
        return mrun

    mrun = lax.fori_loop(0, NCH1 // 2, pair_body,
                         jnp.full((16,), -3.0e38, jnp.float32))
    if NCH1 % 2:
        mrun = process(NCH1 - 1, 0, mrun)
    pltpu.make_async_copy(lbuf.at[0], log_hbm.at[pl.ds(0, C1 * 4)], st[0]).wait()
    pltpu.make_async_copy(lbuf.at[1], log_hbm.at[pl.ds(0, C1 * 4)], st[1]).wait()
    mbuf[...] = mrun
    pltpu.sync_copy(mbuf, max_hbm.at[wid])


_sc_logits = functools.partial(
    pl.kernel,
    out_type=(jax.ShapeDtypeStruct((E * 4,), jnp.float32),
              jax.ShapeDtypeStruct((NW, 16), jnp.float32)),
    mesh=_mesh,
    compiler_params=pltpu.CompilerParams(use_tc_tiling_on_sc=False),
    scratch_types=[
        pltpu.VMEM((NCH1, C1), jnp.int32),
        pltpu.VMEM((NCH1, C1), jnp.int32),
        pltpu.VMEM((2, C1, D), jnp.float32),
        pltpu.VMEM((2, C1, D), jnp.float32),
        pltpu.VMEM((2, C1 * 4), jnp.float32),
        pltpu.VMEM((16,), jnp.float32),
        pltpu.SemaphoreType.DMA,
        pltpu.SemaphoreType.DMA,
        pltpu.SemaphoreType.DMA,
        pltpu.SemaphoreType.DMA,
        pltpu.SemaphoreType.DMA,
        pltpu.SemaphoreType.DMA,
    ],
)(_sc_logits_body)


def _sc_accum_body(ex_hbm, src2_hbm, dst2_hbm, vcat_hbm,
                   cvals_hbm, out_hbm,
                   acc, didx, sidx, didxs, vbufg, vbuf, exbuf, zbuf, cbuf,
                   ix0, ix1, gv0, gv1, ge0, ge1, sc0, sc1):
    c = lax.axis_index("c")
    s = lax.axis_index("s")
    ix = (ix0, ix1)
    gv = (gv0, gv1)
    ge = (ge0, ge1)
    scs = (sc0, sc1)
    pltpu.sync_copy(cvals_hbm.at[c], cbuf)
    c2v = cbuf[...]
    rowoff = c2v * (N // 2)
    iota = _iota16()
    rbase = s * NCH2

    zv = jnp.zeros((16,), jnp.float32)

    def zrow(r, _):
        for j in range(W2 // 16):
            zbuf[r, pl.ds(j * 16, 16)] = zv
        return 0

    lax.fori_loop(0, ZR, zrow, 0)
    for t in range(NSTR // ZR):
        pltpu.sync_copy(zbuf, acc.at[pl.ds(s * NSTR + t * ZR, ZR)])
    plsc.subcore_barrier()

    def idxstart(ci, b):
        pltpu.async_copy(dst2_hbm.at[rbase + ci], didx.at[b], ix[b])
        pltpu.async_copy(src2_hbm.at[rbase + ci], sidx.at[b], ix[b])

    def idxwait(ci, b):
        pltpu.make_async_copy(dst2_hbm.at[rbase + ci], didx.at[b], ix[b]).wait()
        pltpu.make_async_copy(src2_hbm.at[rbase + ci], sidx.at[b], ix[b]).wait()

    def gstart(ci, b):
        for j in range(C2 // 16):
            sidx[b, pl.ds(j * 16, 16)] = sidx[b, pl.ds(j * 16, 16)] + rowoff
        pltpu.async_copy(vcat_hbm.at[sidx.at[b]], vbufg.at[b], gv[b])
        pltpu.async_copy(
            ex_hbm.at[pl.ds((rbase + ci) * (C2 * 4), C2 * 4)],
            exbuf.at[b], ge[b])

    def process(ci, b):
        b2 = 1 - b

        @pl.when(ci + 1 < NCH2)
        def _():
            idxwait(ci + 1, b2)
            gstart(ci + 1, b2)

        pltpu.make_async_copy(
            vcat_hbm.at[sidx.at[b]], vbufg.at[b], gv[b]).wait()
        pltpu.make_async_copy(
            ex_hbm.at[pl.ds((rbase + ci) * (C2 * 4), C2 * 4)],
            exbuf.at[b], ge[b]).wait()

        @pl.when(ci >= 2)
        def _():
            pltpu.make_async_copy(
                vbuf.at[b], acc.at[didxs.at[b]], scs[b]).wait()

        for j in range(C2 // 16):
            didxs[b, pl.ds(j * 16, 16)] = didx[b, pl.ds(j * 16, 16)]

        @pl.when(ci + 2 < NCH2)
        def _():
            idxstart(ci + 2, b)

        def quad_body(jq, _):
            qv = exbuf[b, pl.ds(jq * 16, 16)]
            for r in range(4):
                e = jq * 4 + r
                i0 = jnp.full((16,), 4 * r, jnp.int32) + c2v
                m0 = qv.at[i0].get(mode="promise_in_bounds")
                m1 = qv.at[i0 + 1].get(mode="promise_in_bounds")
                for jv in range(8):
                    g = vbufg[b, e, pl.ds(jv * 16, 16)]
                    vbuf[b, e, pl.ds(jv * 16, 16)] = g * (m0 if jv < 4 else m1)
                tidx = jnp.where(iota < 4, iota + 4 * r, 0)
                tail = jnp.where(
                    iota < 4,
                    qv.at[tidx].get(mode="promise_in_bounds"),
                    0.0)
                vbuf[b, e, pl.ds(128, 16)] = tail
            return 0

        lax.fori_loop(0, C2 // 4, quad_body, 0, unroll=8)
        pltpu.async_copy(vbuf.at[b], acc.at[didxs.at[b]], scs[b], add=True)

    idxstart(0, 0)
    idxstart(1, 1)
    idxwait(0, 0)
    gstart(0, 0)

    def pair_body(t, _):
        process(2 * t, 0)
        process(2 * t + 1, 1)
        return 0

    lax.fori_loop(0, NCH2 // 2, pair_body, 0)
    if NCH2 % 2:
        process(NCH2 - 1, 0)
    pltpu.make_async_copy(vbuf.at[0], acc.at[didxs.at[0]], scs[0]).wait()
    pltpu.make_async_copy(vbuf.at[1], acc.at[didxs.at[1]], scs[1]).wait()
    plsc.subcore_barrier()

    for t in range(NSTR // ZR):
        start = s * NSTR + t * ZR
        pltpu.sync_copy(acc.at[pl.ds(start, ZR)],
                        out_hbm.at[pl.ds(c * N + start, ZR)])


_sc_accum = functools.partial(
    pl.kernel,
    out_type=jax.ShapeDtypeStruct((2 * N, W2), jnp.float32),
    mesh=_mesh,
    compiler_params=pltpu.CompilerParams(use_tc_tiling_on_sc=False),
    scratch_types=[
        pltpu.VMEM_SHARED((N, W2), jnp.float32),
        pltpu.VMEM((2, C2), jnp.int32),
        pltpu.VMEM((2, C2), jnp.int32),
        pltpu.VMEM((2, C2), jnp.int32),
        pltpu.VMEM((2, C2, 128), jnp.float32),
        pltpu.VMEM((2, C2, W2), jnp.float32),
        pltpu.VMEM((2, C2 * 4), jnp.float32),
        pltpu.VMEM((ZR, W2), jnp.float32),
        pltpu.VMEM((16,), jnp.int32),
        pltpu.SemaphoreType.DMA,
        pltpu.SemaphoreType.DMA,
        pltpu.SemaphoreType.DMA,
        pltpu.SemaphoreType.DMA,
        pltpu.SemaphoreType.DMA,
        pltpu.SemaphoreType.DMA,
        pltpu.SemaphoreType.DMA,
        pltpu.SemaphoreType.DMA,
    ],
)(_sc_accum_body)


def _exp_body(l_ref, m_ref, ex_ref):
    M = jnp.max(m_ref[:])
    ex_ref[:] = jnp.exp(l_ref[:] - M)


def _exp_tc(logits, tmax):
    ex = pl.pallas_call(
        _exp_body,
        out_shape=jax.ShapeDtypeStruct((E * 4 // 512, 512), jnp.float32),
    )(logits.reshape(E * 4 // 512, 512), tmax)
    return ex.reshape(E * 4)


_QB = 2000


def _qkv_body(x_ref, wq_ref, bq_ref, wk_ref, bk_ref, wv_ref, bv_ref,
              q_ref, k_ref, vcat_ref):
    x = x_ref[:]
    q_ref[:] = jnp.dot(x, wq_ref[:], preferred_element_type=jnp.float32) + bq_ref[:]
    k_ref[:] = jnp.dot(x, wk_ref[:], preferred_element_type=jnp.float32) + bk_ref[:]
    v = jnp.dot(x, wv_ref[:], preferred_element_type=jnp.float32) + bv_ref[:]
    vcat_ref[0] = v[:, :128]
    vcat_ref[1] = v[:, 128:]


def _qkv(x, Wq, bq, Wk, bk, Wv, bv):
    fi = x.shape[1]
    q, k, v2 = pl.pallas_call(
        _qkv_body,
        grid=(N // _QB,),
        in_specs=[
            pl.BlockSpec((_QB, fi), lambda i: (i, 0)),
            pl.BlockSpec((fi, D), lambda i: (0, 0)),
            pl.BlockSpec((1, D), lambda i: (0, 0)),
            pl.BlockSpec((fi, D), lambda i: (0, 0)),
            pl.BlockSpec((1, D), lambda i: (0, 0)),
            pl.BlockSpec((fi, D), lambda i: (0, 0)),
            pl.BlockSpec((1, D), lambda i: (0, 0)),
        ],
        out_specs=(
            pl.BlockSpec((_QB, D), lambda i: (i, 0)),
            pl.BlockSpec((_QB, D), lambda i: (i, 0)),
            pl.BlockSpec((2, _QB, 128), lambda i: (0, i, 0)),
        ),
        out_shape=(jax.ShapeDtypeStruct((N, D), jnp.float32),
                   jax.ShapeDtypeStruct((N, D), jnp.float32),
                   jax.ShapeDtypeStruct((2, N, 128), jnp.float32)),
    )(x, Wq, bq.reshape(1, D), Wk, bk.reshape(1, D), Wv, bv.reshape(1, D))
    return q, k, v2.reshape(2 * N, 128)


def _post_core(outc, g, beta):
    t = jnp.concatenate([outc[:N, :128], outc[N:, :128]], axis=1)
    den = outc[:N, 128:132]
    den_exp = jnp.concatenate(
        [jnp.broadcast_to(den[:, i:i + 1], (N, 64)) for i in range(4)], axis=1)
    t = jax.nn.relu(t / (den_exp + 1e-16))
    mu = jnp.mean(t, axis=0, keepdims=True)
    var = jnp.mean((t - mu) ** 2, axis=0, keepdims=True)
    return g * (t - mu) / jnp.sqrt(var + 1e-5) + beta


def _post_body(outc_ref, res_ref, g_ref, beta_ref, h_ref, *, use_res):
    h = _post_core(outc_ref[:], g_ref[:], beta_ref[:])
    if use_res:
        h = h + res_ref[:]
    h_ref[:] = h


def _post(outc, res, g, beta, use_res):
    return pl.pallas_call(
        functools.partial(_post_body, use_res=use_res),
        out_shape=jax.ShapeDtypeStruct((N, D), jnp.float32),
    )(outc, res, g.reshape(1, D), beta.reshape(1, D))


def _pool_body(h_ref, ib_ref, pool_ref):
    ib = ib_ref[:]
    gids = lax.broadcasted_iota(jnp.int32, (G, N), 0)
    onehot = (gids == ib).astype(jnp.float32)
    cnt = jnp.sum(onehot, axis=1, keepdims=True)
    h = h_ref[:]
    h_hi = h.astype(jnp.bfloat16).astype(jnp.float32)
    h_lo = h - h_hi
    s = (jnp.dot(onehot, h_hi, preferred_element_type=jnp.float32)
         + jnp.dot(onehot, h_lo, preferred_element_type=jnp.float32))
    pool_ref[:] = s / jnp.maximum(cnt, 1.0)


def _pool(h, ibatch):
    return pl.pallas_call(
        _pool_body,
        out_shape=jax.ShapeDtypeStruct((G, D), jnp.float32),
    )(h, ibatch.reshape(1, N))


def _cline_body(x_ref, wc_ref, bc_ref, w1_ref, b1_ref, w2_ref, b2_ref, o_ref):
    c = jnp.tanh(jnp.dot(x_ref[:], wc_ref[:], preferred_element_type=jnp.float32) + bc_ref[:])
    c = c + jax.nn.relu(jnp.dot(c, w1_ref[:], preferred_element_type=jnp.float32) + b1_ref[:])
    c = c + jax.nn.relu(jnp.dot(c, w2_ref[:], preferred_element_type=jnp.float32) + b2_ref[:])
    o_ref[:] = c


def _cline_mlp(cline_x, Wc, bc, Wc1, bc1, Wc2, bc2):
    return pl.pallas_call(
        _cline_body,
        out_shape=jax.ShapeDtypeStruct((B, D), jnp.float32),
    )(cline_x, Wc, bc.reshape(1, D), Wc1, bc1.reshape(1, D), Wc2, bc2.reshape(1, D))


def kernel(drug_x, drug_adj, ibatch, cline_x, Wq0, Wk0, Wv0, bq0, bk0, bv0, g0, beta0, Wq1, Wk1, Wv1, bq1, bk1, bv1, g1, beta1, Wq2, Wk2, Wv2, bq2, bk2, bv2, g2, beta2, Wc, bc, Wc1, bc1, Wc2, bc2):
    src = drug_adj[0].reshape(E // C1, C1)
    dst = drug_adj[1].reshape(E // C1, C1)
    srcb = drug_adj[0].reshape(E // C2, C2)
    dstb = drug_adj[1].reshape(E // C2, C2)
    cvals = jnp.repeat(jnp.array([[0], [2]], jnp.int32), 16, axis=1)

    q, k, vcat = _qkv(drug_x, Wq0, bq0, Wk0, bk0, Wv0, bv0)
    logits, tmax = _sc_logits(q, k, src, dst)
    outc = _sc_accum(_exp_tc(logits, tmax), srcb, dstb, vcat, cvals)
    h1 = _post(outc, drug_x, g0, beta0, use_res=False)
    q, k, vcat = _qkv(h1, Wq1, bq1, Wk1, bk1, Wv1, bv1)

    logits, tmax = _sc_logits(q, k, src, dst)
    outc = _sc_accum(_exp_tc(logits, tmax), srcb, dstb, vcat, cvals)
    h2 = _post(outc, h1, g1, beta1, use_res=True)
    q, k, vcat = _qkv(h2, Wq2, bq2, Wk2, bk2, Wv2, bv2)

    logits, tmax = _sc_logits(q, k, src, dst)
    outc = _sc_accum(_exp_tc(logits, tmax), srcb, dstb, vcat, cvals)
    h3 = _post(outc, h2, g2, beta2, use_res=True)

    drug_pool = _pool(h3, ibatch)
    c = _cline_mlp(cline_x, Wc, bc, Wc1, bc1, Wc2, bc2)
    return (drug_pool, c)


# speedup vs baseline: 1.5718x; 1.0003x over previous
import functools

import jax
import jax.numpy as jnp
from jax import lax
from jax.experimental import pallas as pl
from jax.experimental.pallas import tpu as pltpu
from jax.experimental.pallas import tpu_sc as plsc

N, E, DIN, D, CIN, B, G, H = 10000, 320000, 128, 256, 512, 1024, 256, 4

NW = 32
C1 = 80
EC1 = E // NW
NCH1 = EC1 // C1
C2 = 32
EC2 = E // 16
NCH2 = EC2 // C2
W2 = 144
NSTR = N // 16
ZR = 125

_mesh = plsc.VectorSubcoreMesh(core_axis_name="c", subcore_axis_name="s")


def _iota16():
    return lax.iota(jnp.int32, 16)


def _lanesum(a):
    iota = _iota16()
    for sh in (8, 4, 2, 1):
        a = a + a.at[iota ^ sh].get(mode="promise_in_bounds")
    return a


def _sc_logits_body(q_hbm, k_hbm, src2_hbm, dst2_hbm, log_hbm, max_hbm,
                    didx_all, sidx_all, qbuf, kbuf, lbuf, mbuf,
                    gq0, gq1, gk0, gk1, st0, st1):
    c = lax.axis_index("c")
    s = lax.axis_index("s")
    wid = s * 2 + c
    rbase = wid * NCH1
    gq = (gq0, gq1)
    gk = (gk0, gk1)
    st = (st0, st1)
    iota = _iota16()

    pltpu.sync_copy(dst2_hbm.at[pl.ds(rbase, NCH1)], didx_all)
    pltpu.sync_copy(src2_hbm.at[pl.ds(rbase, NCH1)], sidx_all)

    def gstart(ci, b):
        pltpu.async_copy(q_hbm.at[didx_all.at[ci]], qbuf.at[b], gq[b])
        pltpu.async_copy(k_hbm.at[sidx_all.at[ci]], kbuf.at[b], gk[b])

    def gwait(ci, b):
        pltpu.make_async_copy(q_hbm.at[didx_all.at[ci]], qbuf.at[b], gq[b]).wait()
        pltpu.make_async_copy(k_hbm.at[sidx_all.at[ci]], kbuf.at[b], gk[b]).wait()

    def process(ci, b, mrun):
        eb4 = (rbase + ci) * (C1 * 4)
        gwait(ci, b)

        @pl.when(ci >= 2)
        def _():
            pltpu.make_async_copy(
                lbuf.at[b], log_hbm.at[pl.ds(eb4, C1 * 4)], st[b]).wait()

        def quad_body(jq, mrun):
            lvec = jnp.zeros((16,), jnp.float32)
            for r in range(4):
                e = jq * 4 + r
                for h in range(H):
                    a = (qbuf[b, e, pl.ds(h * 64, 16)]
                         * kbuf[b, e, pl.ds(h * 64, 16)])
                    for j in range(1, 4):
                        o = h * 64 + j * 16
                        a = a + (qbuf[b, e, pl.ds(o, 16)]
                                 * kbuf[b, e, pl.ds(o, 16)])
                    sv = _lanesum(a) * 0.125
                    lvec = jnp.where(iota == 4 * r + h, sv, lvec)
            lbuf[b, pl.ds(jq * 16, 16)] = lvec
            return jnp.maximum(mrun, lvec)

        mrun = lax.fori_loop(0, C1 // 4, quad_body, mrun, unroll=4)
        pltpu.async_copy(lbuf.at[b], log_hbm.at[pl.ds(eb4, C1 * 4)], st[b])

        @pl.when(ci + 2 < NCH1)
        def _():
            gstart(ci + 2, b)

        return mrun

    gstart(0, 0)
    gstart(1, 1)

    def pair_body(t, mrun):
        mrun = process(2 * t, 0, mrun)
        mrun = process(2 * t + 1, 1, mrun)
        return mrun

    mrun = lax.fori_loop(0, NCH1 // 2, pair_body,
                         jnp.full((16,), -3.0e38, jnp.float32))
    if NCH1 % 2:
        mrun = process(NCH1 - 1, 0, mrun)
    pltpu.make_async_copy(lbuf.at[0], log_hbm.at[pl.ds(0, C1 * 4)], st[0]).wait()
    pltpu.make_async_copy(lbuf.at[1], log_hbm.at[pl.ds(0, C1 * 4)], st[1]).wait()
    mbuf[...] = mrun
    pltpu.sync_copy(mbuf, max_hbm.at[wid])


_sc_logits = functools.partial(
    pl.kernel,
    out_type=(jax.ShapeDtypeStruct((E * 4,), jnp.float32),
              jax.ShapeDtypeStruct((NW, 16), jnp.float32)),
    mesh=_mesh,
    compiler_params=pltpu.CompilerParams(use_tc_tiling_on_sc=False),
    scratch_types=[
        pltpu.VMEM((NCH1, C1), jnp.int32),
        pltpu.VMEM((NCH1, C1), jnp.int32),
        pltpu.VMEM((2, C1, D), jnp.float32),
        pltpu.VMEM((2, C1, D), jnp.float32),
        pltpu.VMEM((2, C1 * 4), jnp.float32),
        pltpu.VMEM((16,), jnp.float32),
        pltpu.SemaphoreType.DMA,
        pltpu.SemaphoreType.DMA,
        pltpu.SemaphoreType.DMA,
        pltpu.SemaphoreType.DMA,
        pltpu.SemaphoreType.DMA,
        pltpu.SemaphoreType.DMA,
    ],
)(_sc_logits_body)


def _sc_accum_body(ex_hbm, src2_hbm, dst2_hbm, vcat_hbm,
                   cvals_hbm, out_hbm,
                   acc, didx, sidx, didxs, vbufg, vbuf, exbuf, zbuf, cbuf,
                   ix0, ix1, gv0, gv1, ge0, ge1, sc0, sc1):
    c = lax.axis_index("c")
    s = lax.axis_index("s")
    ix = (ix0, ix1)
    gv = (gv0, gv1)
    ge = (ge0, ge1)
    scs = (sc0, sc1)
    pltpu.sync_copy(cvals_hbm.at[c], cbuf)
    c2v = cbuf[...]
    rowoff = c2v * (N // 2)
    iota = _iota16()
    rbase = s * NCH2

    zv = jnp.zeros((16,), jnp.float32)

    def zrow(r, _):
        for j in range(W2 // 16):
            zbuf[r, pl.ds(j * 16, 16)] = zv
        return 0

    lax.fori_loop(0, ZR, zrow, 0)
    for t in range(NSTR // ZR):
        pltpu.sync_copy(zbuf, acc.at[pl.ds(s * NSTR + t * ZR, ZR)])
    plsc.subcore_barrier()

    def idxstart(ci, b):
        pltpu.async_copy(dst2_hbm.at[rbase + ci], didx.at[b], ix[b])
        pltpu.async_copy(src2_hbm.at[rbase + ci], sidx.at[b], ix[b])

    def idxwait(ci, b):
        pltpu.make_async_copy(dst2_hbm.at[rbase + ci], didx.at[b], ix[b]).wait()
        pltpu.make_async_copy(src2_hbm.at[rbase + ci], sidx.at[b], ix[b]).wait()

    def gstart(ci, b):
        for j in range(C2 // 16):
            sidx[b, pl.ds(j * 16, 16)] = sidx[b, pl.ds(j * 16, 16)] + rowoff
        pltpu.async_copy(vcat_hbm.at[sidx.at[b]], vbufg.at[b], gv[b])
        pltpu.async_copy(
            ex_hbm.at[pl.ds((rbase + ci) * (C2 * 4), C2 * 4)],
            exbuf.at[b], ge[b])

    def process(ci, b):
        b2 = 1 - b

        @pl.when(ci + 1 < NCH2)
        def _():
            idxwait(ci + 1, b2)
            gstart(ci + 1, b2)

        pltpu.make_async_copy(
            vcat_hbm.at[sidx.at[b]], vbufg.at[b], gv[b]).wait()
        pltpu.make_async_copy(
            ex_hbm.at[pl.ds((rbase + ci) * (C2 * 4), C2 * 4)],
            exbuf.at[b], ge[b]).wait()

        @pl.when(ci >= 2)
        def _():
            pltpu.make_async_copy(
                vbuf.at[b], acc.at[didxs.at[b]], scs[b]).wait()

        for j in range(C2 // 16):
            didxs[b, pl.ds(j * 16, 16)] = didx[b, pl.ds(j * 16, 16)]

        @pl.when(ci + 2 < NCH2)
        def _():
            idxstart(ci + 2, b)

        def quad_body(jq, _):
            qv = exbuf[b, pl.ds(jq * 16, 16)]
            for r in range(4):
                e = jq * 4 + r
                i0 = jnp.full((16,), 4 * r, jnp.int32) + c2v
                m0 = qv.at[i0].get(mode="promise_in_bounds")
                m1 = qv.at[i0 + 1].get(mode="promise_in_bounds")
                for jv in range(8):
                    g = vbufg[b, e, pl.ds(jv * 16, 16)]
                    vbuf[b, e, pl.ds(jv * 16, 16)] = g * (m0 if jv < 4 else m1)
                tidx = jnp.where(iota < 4, iota + 4 * r, 0)
                tail = jnp.where(
                    iota < 4,
                    qv.at[tidx].get(mode="promise_in_bounds"),
                    0.0)
                vbuf[b, e, pl.ds(128, 16)] = tail
            return 0

        lax.fori_loop(0, C2 // 4, quad_body, 0, unroll=8)
        pltpu.async_copy(vbuf.at[b], acc.at[didxs.at[b]], scs[b], add=True)

    idxstart(0, 0)
    idxstart(1, 1)
    idxwait(0, 0)
    gstart(0, 0)

    def pair_body(t, _):
        process(2 * t, 0)
        process(2 * t + 1, 1)
        return 0

    lax.fori_loop(0, NCH2 // 2, pair_body, 0)
    if NCH2 % 2:
        process(NCH2 - 1, 0)
    pltpu.make_async_copy(vbuf.at[0], acc.at[didxs.at[0]], scs[0]).wait()
    pltpu.make_async_copy(vbuf.at[1], acc.at[didxs.at[1]], scs[1]).wait()
    plsc.subcore_barrier()

    for t in range(NSTR // ZR):
        start = s * NSTR + t * ZR
        pltpu.sync_copy(acc.at[pl.ds(start, ZR)],
                        out_hbm.at[pl.ds(c * N + start, ZR)])


_sc_accum = functools.partial(
    pl.kernel,
    out_type=jax.ShapeDtypeStruct((2 * N, W2), jnp.float32),
    mesh=_mesh,
    compiler_params=pltpu.CompilerParams(use_tc_tiling_on_sc=False),
    scratch_types=[
        pltpu.VMEM_SHARED((N, W2), jnp.float32),
        pltpu.VMEM((2, C2), jnp.int32),
        pltpu.VMEM((2, C2), jnp.int32),
        pltpu.VMEM((2, C2), jnp.int32),
        pltpu.VMEM((2, C2, 128), jnp.float32),
        pltpu.VMEM((2, C2, W2), jnp.float32),
        pltpu.VMEM((2, C2 * 4), jnp.float32),
        pltpu.VMEM((ZR, W2), jnp.float32),
        pltpu.VMEM((16,), jnp.int32),
        pltpu.SemaphoreType.DMA,
        pltpu.SemaphoreType.DMA,
        pltpu.SemaphoreType.DMA,
        pltpu.SemaphoreType.DMA,
        pltpu.SemaphoreType.DMA,
        pltpu.SemaphoreType.DMA,
        pltpu.SemaphoreType.DMA,
        pltpu.SemaphoreType.DMA,
    ],
)(_sc_accum_body)


def _exp_body(l_ref, m_ref, ex_ref):
    M = jnp.max(m_ref[:])
    ex_ref[:] = jnp.exp(l_ref[:] - M)


def _exp_tc(logits, tmax):
    ex = pl.pallas_call(
        _exp_body,
        out_shape=jax.ShapeDtypeStruct((E * 4 // 512, 512), jnp.float32),
    )(logits.reshape(E * 4 // 512, 512), tmax)
    return ex.reshape(E * 4)


_QB = 2000


def _qkv_body(x_ref, wq_ref, bq_ref, wk_ref, bk_ref, wv_ref, bv_ref,
              q_ref, k_ref, vcat_ref):
    x = x_ref[:]
    q_ref[:] = jnp.dot(x, wq_ref[:], preferred_element_type=jnp.float32) + bq_ref[:]
    k_ref[:] = jnp.dot(x, wk_ref[:], preferred_element_type=jnp.float32) + bk_ref[:]
    v = jnp.dot(x, wv_ref[:], preferred_element_type=jnp.float32) + bv_ref[:]
    vcat_ref[0] = v[:, :128]
    vcat_ref[1] = v[:, 128:]


def _qkv(x, Wq, bq, Wk, bk, Wv, bv):
    fi = x.shape[1]
    q, k, v2 = pl.pallas_call(
        _qkv_body,
        grid=(N // _QB,),
        in_specs=[
            pl.BlockSpec((_QB, fi), lambda i: (i, 0)),
            pl.BlockSpec((fi, D), lambda i: (0, 0)),
            pl.BlockSpec((1, D), lambda i: (0, 0)),
            pl.BlockSpec((fi, D), lambda i: (0, 0)),
            pl.BlockSpec((1, D), lambda i: (0, 0)),
            pl.BlockSpec((fi, D), lambda i: (0, 0)),
            pl.BlockSpec((1, D), lambda i: (0, 0)),
        ],
        out_specs=(
            pl.BlockSpec((_QB, D), lambda i: (i, 0)),
            pl.BlockSpec((_QB, D), lambda i: (i, 0)),
            pl.BlockSpec((2, _QB, 128), lambda i: (0, i, 0)),
        ),
        out_shape=(jax.ShapeDtypeStruct((N, D), jnp.float32),
                   jax.ShapeDtypeStruct((N, D), jnp.float32),
                   jax.ShapeDtypeStruct((2, N, 128), jnp.float32)),
    )(x, Wq, bq.reshape(1, D), Wk, bk.reshape(1, D), Wv, bv.reshape(1, D))
    return q, k, v2.reshape(2 * N, 128)


def _post_core(outc, g, beta):
    t = jnp.concatenate([outc[:N, :128], outc[N:, :128]], axis=1)
    den = outc[:N, 128:132]
    den_exp = jnp.concatenate(
        [jnp.broadcast_to(den[:, i:i + 1], (N, 64)) for i in range(4)], axis=1)
    t = jax.nn.relu(t / (den_exp + 1e-16))
    mu = jnp.mean(t, axis=0, keepdims=True)
    var = jnp.mean((t - mu) ** 2, axis=0, keepdims=True)
    return g * (t - mu) / jnp.sqrt(var + 1e-5) + beta


def _post_body(outc_ref, res_ref, g_ref, beta_ref, h_ref, *, use_res):
    h = _post_core(outc_ref[:], g_ref[:], beta_ref[:])
    if use_res:
        h = h + res_ref[:]
    h_ref[:] = h


def _post(outc, res, g, beta, use_res):
    return pl.pallas_call(
        functools.partial(_post_body, use_res=use_res),
        out_shape=jax.ShapeDtypeStruct((N, D), jnp.float32),
    )(outc, res, g.reshape(1, D), beta.reshape(1, D))


def _pool_body(h_ref, ib_ref, pool_ref):
    ib = ib_ref[:]
    gids = lax.broadcasted_iota(jnp.int32, (G, N), 0)
    onehot = (gids == ib).astype(jnp.float32)
    cnt = jnp.sum(onehot, axis=1, keepdims=True)
    h = h_ref[:]
    h_hi = h.astype(jnp.bfloat16).astype(jnp.float32)
    h_lo = h - h_hi
    s = (jnp.dot(onehot, h_hi, preferred_element_type=jnp.float32)
         + jnp.dot(onehot, h_lo, preferred_element_type=jnp.float32))
    pool_ref[:] = s / jnp.maximum(cnt, 1.0)


def _pool(h, ibatch):
    return pl.pallas_call(
        _pool_body,
        out_shape=jax.ShapeDtypeStruct((G, D), jnp.float32),
    )(h, ibatch.reshape(1, N))


def _cline_body(x_ref, wc_ref, bc_ref, w1_ref, b1_ref, w2_ref, b2_ref, o_ref):
    c = jnp.tanh(jnp.dot(x_ref[:], wc_ref[:], preferred_element_type=jnp.float32) + bc_ref[:])
    c = c + jax.nn.relu(jnp.dot(c, w1_ref[:], preferred_element_type=jnp.float32) + b1_ref[:])
    c = c + jax.nn.relu(jnp.dot(c, w2_ref[:], preferred_element_type=jnp.float32) + b2_ref[:])
    o_ref[:] = c


def _cline_mlp(cline_x, Wc, bc, Wc1, bc1, Wc2, bc2):
    return pl.pallas_call(
        _cline_body,
        out_shape=jax.ShapeDtypeStruct((B, D), jnp.float32),
    )(cline_x, Wc, bc.reshape(1, D), Wc1, bc1.reshape(1, D), Wc2, bc2.reshape(1, D))


def kernel(drug_x, drug_adj, ibatch, cline_x, Wq0, Wk0, Wv0, bq0, bk0, bv0, g0, beta0, Wq1, Wk1, Wv1, bq1, bk1, bv1, g1, beta1, Wq2, Wk2, Wv2, bq2, bk2, bv2, g2, beta2, Wc, bc, Wc1, bc1, Wc2, bc2):
    src = drug_adj[0].reshape(E // C1, C1)
    dst = drug_adj[1].reshape(E // C1, C1)
    srcb = drug_adj[0].reshape(E // C2, C2)
    dstb = drug_adj[1].reshape(E // C2, C2)
    cvals = jnp.repeat(jnp.array([[0], [2]], jnp.int32), 16, axis=1)

    q, k, vcat = _qkv(drug_x, Wq0, bq0, Wk0, bk0, Wv0, bv0)
    logits, tmax = _sc_logits(q, k, src, dst)
    outc = _sc_accum(_exp_tc(logits, tmax), srcb, dstb, vcat, cvals)
    h1 = _post(outc, drug_x, g0, beta0, use_res=False)
    q, k, vcat = _qkv(h1, Wq1, bq1, Wk1, bk1, Wv1, bv1)

    logits, tmax = _sc_logits(q, k, src, dst)
    outc = _sc_accum(_exp_tc(logits, tmax), srcb, dstb, vcat, cvals)
    h2 = _post(outc, h1, g1, beta1, use_res=True)
    q, k, vcat = _qkv(h2, Wq2, bq2, Wk2, bk2, Wv2, bv2)

    logits, tmax = _sc_logits(q, k, src, dst)
    outc = _sc_accum(_exp_tc(logits, tmax), srcb, dstb, vcat, cvals)
    h3 = _post(outc, h2, g2, beta2, use_res=True)

    drug_pool = _pool(h3, ibatch)
    c = _cline_mlp(cline_x, Wc, bc, Wc1, bc1, Wc2, bc2)
    return (drug_pool, c)
